# baseline (speedup 1.0000x reference)
import functools

import jax
import jax.numpy as jnp
from jax import lax
from jax.experimental import pallas as pl
from jax.experimental.pallas import tpu as pltpu
from jax.experimental.pallas import tpu_sc as plsc

N = 10000
D = 128
H = 128
E = 320000

NC = 2
NS = 16
CHUNK = 128
N0_CH = 111
N1_CH = 46
TOTCH = NS * (N0_CH + N1_CH)
E_PAD = TOTCH * CHUNK
NP = 10016
ROWS_PER_TILE = NP // NS
CW = 8

BLK = 400
GRID = N // BLK


def _ln(h, g, b):
    m = jnp.mean(h, axis=-1, keepdims=True)
    v = jnp.mean((h - m) * (h - m), axis=-1, keepdims=True)
    return (h - m) / jnp.sqrt(v + 1e-5) * g + b


def _proj_body(x_ref, nt_ref, pg, pbta, pwt, pbi, fg, fb, fwt, fbi,
               sg, sb, swt, sbi, emb_ref, out_ref):
    x = jnp.clip(x_ref[...], -10.0, 10.0)
    p = jnp.dot(_ln(x, pg[...], pbta[...]), pwt[...],
                preferred_element_type=jnp.float32) + pbi[...]
    f = jnp.dot(_ln(x, fg[...], fb[...]), fwt[...],
                preferred_element_type=jnp.float32) + fbi[...]
    s = jnp.dot(_ln(x, sg[...], sb[...]), swt[...],
                preferred_element_type=jnp.float32) + sbi[...]
    nt = nt_ref[...]
    sel = jnp.where(nt == 0, p, jnp.where(nt == 1, f,
                    jnp.where(nt == 2, s, 0.0)))
    te = jnp.where(nt == 0, emb_ref[0:1, :], jnp.where(
        nt == 1, emb_ref[1:2, :], emb_ref[2:3, :]))
    out_ref[...] = sel + te


def _proj(x, nt2, pg, pbta, pwt, pbi, fg, fb, fwt, fbi, sg, sb, swt, sbi, emb):
    row = lambda i: (i, 0)
    full = lambda i: (0, 0)
    vec = pl.BlockSpec((1, H), full)
    return pl.pallas_call(
        _proj_body,
        grid=(GRID,),
        in_specs=[
            pl.BlockSpec((BLK, D), row),
            pl.BlockSpec((BLK, 1), row),
            vec, vec, pl.BlockSpec((D, H), full), vec,
            vec, vec, pl.BlockSpec((D, H), full), vec,
            vec, vec, pl.BlockSpec((D, H), full), vec,
            pl.BlockSpec((8, H), full),
        ],
        out_specs=pl.BlockSpec((BLK, H), row),
        out_shape=jax.ShapeDtypeStruct((N, H), jnp.float32),
    )(x, nt2, pg, pbta, pwt, pbi, fg, fb, fwt, fbi, sg, sb, swt, sbi, emb)


def _agg_body(h_hbm, src_hbm, dst_hbm, zero_hbm, ones_hbm, out_hbm, cnt_hbm,
              srcs, dsts, bufs, ones_v, acc_sh, cnt_sh,
              isems, gsems, ssems, semc, *, with_cnt):
    c = lax.axis_index("c")
    s = lax.axis_index("s")
    nch = jnp.where(c == 0, N0_CH, N1_CH)
    cbase = c * NS * N0_CH + s * nch

    base = s * ROWS_PER_TILE
    pltpu.sync_copy(zero_hbm.at[pl.ds(0, ROWS_PER_TILE)],
                    acc_sh.at[pl.ds(base, ROWS_PER_TILE)])
    if with_cnt:
        pltpu.sync_copy(zero_hbm.at[pl.ds(0, ROWS_PER_TILE), pl.ds(0, CW)],
                        cnt_sh.at[pl.ds(base, ROWS_PER_TILE)])
        pltpu.sync_copy(ones_hbm, ones_v)
    plsc.subcore_barrier()

    def start_idx(i, q):
        pltpu.async_copy(src_hbm.at[cbase + i], srcs[q], isems[q])
        pltpu.async_copy(dst_hbm.at[cbase + i], dsts[q], isems[q])

    def wait_idx(i, q):
        pltpu.make_async_copy(src_hbm.at[cbase + i], srcs[q], isems[q]).wait()
        pltpu.make_async_copy(dst_hbm.at[cbase + i], dsts[q], isems[q]).wait()

    def start_gather(q, b):
        pltpu.async_copy(h_hbm.at[srcs[q]], bufs[b], gsems[b])

    def wait_gather(q, b):
        pltpu.make_async_copy(h_hbm.at[srcs[q]], bufs[b], gsems[b]).wait()

    def start_scat(q, b):
        pltpu.async_copy(bufs[b], acc_sh.at[dsts[q]], ssems[b], add=True)

    def wait_scat(q, b):
        pltpu.make_async_copy(bufs[b], acc_sh.at[dsts[q]], ssems[b]).wait()

    pltpu.sync_copy(src_hbm.at[cbase + 0], srcs[0])
    pltpu.sync_copy(dst_hbm.at[cbase + 0], dsts[0])
    start_idx(1, 1)
    start_idx(2, 2)
    start_idx(3, 3)
    start_gather(0, 0)

    def body(k, _):
        for j in range(4):
            i = 4 * k + j
            q = j % 4
            b = j % 2
            qp = (j - 1) % 4
            bp = (j - 1) % 2
            @pl.when(i < nch)
            def _():
                wait_gather(q, b)
                start_scat(q, b)
                if with_cnt:
                    @pl.when(i >= 1)
                    def _():
                        pltpu.make_async_copy(
                            ones_v, cnt_sh.at[dsts[qp]], semc).wait()
                    pltpu.async_copy(ones_v, cnt_sh.at[dsts[q]], semc,
                                     add=True)
            @pl.when((i >= 1) & (i - 1 < nch))
            def _():
                wait_scat(qp, bp)
                @pl.when(i + 3 < nch)
                def _():
                    start_idx(i + 3, qp)
            @pl.when(i + 1 < nch)
            def _():
                q2 = (j + 1) % 4
                wait_idx(i + 1, q2)
                start_gather(q2, (j + 1) % 2)
        return 0
    lax.fori_loop(0, (nch + 4) // 4, body, 0)

    if with_cnt:
        pltpu.make_async_copy(ones_v, cnt_sh.at[dsts[0]], semc).wait()

    plsc.subcore_barrier()
    pltpu.sync_copy(acc_sh.at[pl.ds(base, ROWS_PER_TILE)],
                    out_hbm.at[c, pl.ds(base, ROWS_PER_TILE)])
    if with_cnt:
        pltpu.sync_copy(cnt_sh.at[pl.ds(base, ROWS_PER_TILE)],
                        cnt_hbm.at[c, pl.ds(base, ROWS_PER_TILE)])


def _aggregate(h, src2, dst2, zrows, ones_arr, with_cnt):
    mesh = plsc.VectorSubcoreMesh(core_axis_name="c", subcore_axis_name="s",
                                  num_cores=NC, num_subcores=NS)
    out_type = (jax.ShapeDtypeStruct((NC, NP, H), jnp.float32),
                jax.ShapeDtypeStruct((NC, NP, CW), jnp.float32))
    idx_t = pltpu.VMEM((CHUNK,), jnp.int32)
    buf_t = pltpu.VMEM((CHUNK, H), jnp.float32)
    sem = pltpu.SemaphoreType.DMA
    return pl.kernel(
        functools.partial(_agg_body, with_cnt=with_cnt),
        out_type=out_type,
        mesh=mesh,
        compiler_params=pltpu.CompilerParams(use_tc_tiling_on_sc=False),
        scratch_types=[
            (idx_t,) * 4,
            (idx_t,) * 4,
            (buf_t,) * 2,
            pltpu.VMEM((CHUNK, CW), jnp.float32),
            pltpu.VMEM_SHARED((NP, H), jnp.float32),
            pltpu.VMEM_SHARED((NP, CW), jnp.float32),
            (sem,) * 4,
            (sem,) * 2,
            (sem,) * 2,
            sem,
        ],
    )(h, src2, dst2, zrows, ones_arr)


def _layer_body(parts_ref, cnt_ref, h_ref, wlt, bl, wrt, g, b, out_ref):
    sums = parts_ref[0] + parts_ref[1]
    cnt = cnt_ref[0][:, 0:1] + cnt_ref[1][:, 0:1]
    agg = sums / jnp.maximum(cnt, 1.0)
    h = h_ref[...]
    t = (jnp.dot(agg, wlt[...], preferred_element_type=jnp.float32)
         + jnp.dot(h, wrt[...], preferred_element_type=jnp.float32)
         + bl[...] + h)
    out_ref[...] = jax.nn.relu(_ln(t, g[...], b[...]))


def _layer(parts, cnt, h, wlt, bl, wrt, g, b):
    row = lambda i: (i, 0)
    full = lambda i: (0, 0)
    vec = pl.BlockSpec((1, H), full)
    return pl.pallas_call(
        _layer_body,
        grid=(GRID,),
        in_specs=[
            pl.BlockSpec((NC, BLK, H), lambda i: (0, i, 0)),
            pl.BlockSpec((NC, BLK, CW), lambda i: (0, i, 0)),
            pl.BlockSpec((BLK, H), row),
            pl.BlockSpec((H, H), full), vec,
            pl.BlockSpec((H, H), full), vec, vec,
        ],
        out_specs=pl.BlockSpec((BLK, H), row),
        out_shape=jax.ShapeDtypeStruct((N, H), jnp.float32),
    )(parts, cnt, h, wlt, bl, wrt, g, b)


def kernel(x, edge_index, node_type,
           proc_ln_g, proc_ln_b, proc_w, proc_b,
           file_ln_g, file_ln_b, file_w, file_b,
           sock_ln_g, sock_ln_b, sock_w, sock_b,
           type_emb,
           w_l0, b_l0, w_r0, ln_g0, ln_b0,
           w_l1, b_l1, w_r1, ln_g1, ln_b1):
    f32 = jnp.float32
    nt2 = node_type.reshape(N, 1).astype(jnp.int32)
    emb = jnp.zeros((8, H), f32).at[0:3].set(type_emb)
    r1 = lambda v: v.reshape(1, -1).astype(f32)

    h0 = _proj(x, nt2,
               r1(proc_ln_g), r1(proc_ln_b), proc_w.T, r1(proc_b),
               r1(file_ln_g), r1(file_ln_b), file_w.T, r1(file_b),
               r1(sock_ln_g), r1(sock_ln_b), sock_w.T, r1(sock_b),
               emb)

    src = edge_index[0].astype(jnp.int32)
    dst = edge_index[1].astype(jnp.int32)
    pad = E_PAD - E
    src2 = jnp.concatenate([src, jnp.zeros((pad,), jnp.int32)]
                           ).reshape(TOTCH, CHUNK)
    dst2 = jnp.concatenate([dst, jnp.full((pad,), N, jnp.int32)]
                           ).reshape(TOTCH, CHUNK)
    zrows = jnp.zeros((ROWS_PER_TILE, H), f32)
    ones_arr = jnp.zeros((CHUNK, CW), f32).at[:, 0].set(1.0)

    parts0, cnt0 = _aggregate(h0, src2, dst2, zrows, ones_arr, with_cnt=True)
    h1 = _layer(parts0, cnt0, h0, w_l0.T, r1(b_l0), w_r0.T,
                r1(ln_g0), r1(ln_b0))
    parts1, _ = _aggregate(h1, src2, dst2, zrows, ones_arr, with_cnt=False)
    h2 = _layer(parts1, cnt0, h1, w_l1.T, r1(b_l1), w_r1.T,
                r1(ln_g1), r1(ln_b1))
    return h2


# speedup vs baseline: 1.1500x; 1.1500x over previous
import functools

import jax
import jax.numpy as jnp
from jax import lax
from jax.experimental import pallas as pl
from jax.experimental.pallas import tpu as pltpu
from jax.experimental.pallas import tpu_sc as plsc

N = 10000
D = 128
H = 128
E = 320000

NC = 2
NS = 16
CHUNK = 128
N0_CH = 111
N1_CH = 46
TOTCH = NS * (N0_CH + N1_CH)
E_PAD = TOTCH * CHUNK
NP = 10016
ROWS_PER_TILE = NP // NS
CW = 8

BLK = 400
GRID = N // BLK


def _ln(h, g, b):
    m = jnp.mean(h, axis=-1, keepdims=True)
    v = jnp.mean((h - m) * (h - m), axis=-1, keepdims=True)
    return (h - m) / jnp.sqrt(v + 1e-5) * g + b


def _proj_body(x_ref, nt_ref, pg, pbta, pwt, pbi, fg, fb, fwt, fbi,
               sg, sb, swt, sbi, emb_ref, out_ref):
    x = jnp.clip(x_ref[...], -10.0, 10.0)
    p = jnp.dot(_ln(x, pg[...], pbta[...]), pwt[...],
                preferred_element_type=jnp.float32) + pbi[...]
    f = jnp.dot(_ln(x, fg[...], fb[...]), fwt[...],
                preferred_element_type=jnp.float32) + fbi[...]
    s = jnp.dot(_ln(x, sg[...], sb[...]), swt[...],
                preferred_element_type=jnp.float32) + sbi[...]
    nt = nt_ref[...]
    sel = jnp.where(nt == 0, p, jnp.where(nt == 1, f,
                    jnp.where(nt == 2, s, 0.0)))
    te = jnp.where(nt == 0, emb_ref[0:1, :], jnp.where(
        nt == 1, emb_ref[1:2, :], emb_ref[2:3, :]))
    out_ref[...] = sel + te


def _proj(x, nt2, pg, pbta, pwt, pbi, fg, fb, fwt, fbi, sg, sb, swt, sbi, emb):
    row = lambda i: (i, 0)
    full = lambda i: (0, 0)
    vec = pl.BlockSpec((1, H), full)
    return pl.pallas_call(
        _proj_body,
        grid=(GRID,),
        in_specs=[
            pl.BlockSpec((BLK, D), row),
            pl.BlockSpec((BLK, 1), row),
            vec, vec, pl.BlockSpec((D, H), full), vec,
            vec, vec, pl.BlockSpec((D, H), full), vec,
            vec, vec, pl.BlockSpec((D, H), full), vec,
            pl.BlockSpec((8, H), full),
        ],
        out_specs=pl.BlockSpec((BLK, H), row),
        out_shape=jax.ShapeDtypeStruct((N, H), jnp.float32),
    )(x, nt2, pg, pbta, pwt, pbi, fg, fb, fwt, fbi, sg, sb, swt, sbi, emb)


def _agg_body(h_hbm, src_hbm, dst_hbm, zero_hbm, ones_hbm, out_hbm, cnt_hbm,
              srcs, dsts, bufs, ones_v, acc_sh, cnt_sh,
              isems, gsems, ssems, semc, *, with_cnt):
    c = lax.axis_index("c")
    s = lax.axis_index("s")
    nch = jnp.where(c == 0, N0_CH, N1_CH)
    cbase = c * NS * N0_CH + s * nch

    base = s * ROWS_PER_TILE
    pltpu.sync_copy(zero_hbm.at[pl.ds(0, ROWS_PER_TILE)],
                    acc_sh.at[pl.ds(base, ROWS_PER_TILE)])
    if with_cnt:
        pltpu.sync_copy(zero_hbm.at[pl.ds(0, ROWS_PER_TILE), pl.ds(0, CW)],
                        cnt_sh.at[pl.ds(base, ROWS_PER_TILE)])
        pltpu.sync_copy(ones_hbm, ones_v)
    plsc.subcore_barrier()

    def start_idx(i, q):
        pltpu.async_copy(src_hbm.at[cbase + i], srcs[q], isems[q])
        pltpu.async_copy(dst_hbm.at[cbase + i], dsts[q], isems[q])

    def wait_idx(i, q):
        pltpu.make_async_copy(src_hbm.at[cbase + i], srcs[q], isems[q]).wait()
        pltpu.make_async_copy(dst_hbm.at[cbase + i], dsts[q], isems[q]).wait()

    def start_gather(q, b):
        pltpu.async_copy(h_hbm.at[srcs[q]], bufs[b], gsems[b])

    def wait_gather(q, b):
        pltpu.make_async_copy(h_hbm.at[srcs[q]], bufs[b], gsems[b]).wait()

    def start_scat(q, b):
        pltpu.async_copy(bufs[b], acc_sh.at[dsts[q]], ssems[b], add=True)

    def wait_scat(q, b):
        pltpu.make_async_copy(bufs[b], acc_sh.at[dsts[q]], ssems[b]).wait()

    pltpu.sync_copy(src_hbm.at[cbase + 0], srcs[0])
    pltpu.sync_copy(dst_hbm.at[cbase + 0], dsts[0])
    pltpu.sync_copy(src_hbm.at[cbase + 1], srcs[1])
    pltpu.sync_copy(dst_hbm.at[cbase + 1], dsts[1])
    start_idx(2, 2)
    start_idx(3, 3)
    start_gather(0, 0)
    start_gather(1, 1)

    def body(k, _):
        for j in range(4):
            i = 4 * k + j
            q = j
            b = j % 2
            @pl.when(i < nch)
            def _():
                wait_gather(q, b)
                if with_cnt:
                    pltpu.async_copy(ones_v, cnt_sh.at[dsts[q]], semc,
                                     add=True)
                pltpu.sync_copy(bufs[b], acc_sh.at[dsts[q]], add=True)
                if with_cnt:
                    pltpu.make_async_copy(ones_v, cnt_sh.at[dsts[q]],
                                          semc).wait()
                @pl.when(i + 4 < nch)
                def _():
                    start_idx(i + 4, q)
                @pl.when(i + 2 < nch)
                def _():
                    q2 = (j + 2) % 4
                    wait_idx(i + 2, q2)
                    start_gather(q2, b)
        return 0
    lax.fori_loop(0, (nch + 3) // 4, body, 0)

    plsc.subcore_barrier()
    pltpu.sync_copy(acc_sh.at[pl.ds(base, ROWS_PER_TILE)],
                    out_hbm.at[c, pl.ds(base, ROWS_PER_TILE)])
    if with_cnt:
        pltpu.sync_copy(cnt_sh.at[pl.ds(base, ROWS_PER_TILE)],
                        cnt_hbm.at[c, pl.ds(base, ROWS_PER_TILE)])


def _aggregate(h, src2, dst2, zrows, ones_arr, with_cnt):
    mesh = plsc.VectorSubcoreMesh(core_axis_name="c", subcore_axis_name="s",
                                  num_cores=NC, num_subcores=NS)
    out_type = (jax.ShapeDtypeStruct((NC, NP, H), jnp.float32),
                jax.ShapeDtypeStruct((NC, NP, CW), jnp.float32))
    idx_t = pltpu.VMEM((CHUNK,), jnp.int32)
    buf_t = pltpu.VMEM((CHUNK, H), jnp.float32)
    sem = pltpu.SemaphoreType.DMA
    return pl.kernel(
        functools.partial(_agg_body, with_cnt=with_cnt),
        out_type=out_type,
        mesh=mesh,
        compiler_params=pltpu.CompilerParams(use_tc_tiling_on_sc=False),
        scratch_types=[
            (idx_t,) * 4,
            (idx_t,) * 4,
            (buf_t,) * 2,
            pltpu.VMEM((CHUNK, CW), jnp.float32),
            pltpu.VMEM_SHARED((NP, H), jnp.float32),
            pltpu.VMEM_SHARED((NP, CW), jnp.float32),
            (sem,) * 4,
            (sem,) * 2,
            (sem,) * 2,
            sem,
        ],
    )(h, src2, dst2, zrows, ones_arr)


def _layer_body(parts_ref, cnt_ref, h_ref, wlt, bl, wrt, g, b, out_ref):
    sums = parts_ref[0] + parts_ref[1]
    cnt = cnt_ref[0][:, 0:1] + cnt_ref[1][:, 0:1]
    agg = sums / jnp.maximum(cnt, 1.0)
    h = h_ref[...]
    t = (jnp.dot(agg, wlt[...], preferred_element_type=jnp.float32)
         + jnp.dot(h, wrt[...], preferred_element_type=jnp.float32)
         + bl[...] + h)
    out_ref[...] = jax.nn.relu(_ln(t, g[...], b[...]))


def _layer(parts, cnt, h, wlt, bl, wrt, g, b):
    row = lambda i: (i, 0)
    full = lambda i: (0, 0)
    vec = pl.BlockSpec((1, H), full)
    return pl.pallas_call(
        _layer_body,
        grid=(GRID,),
        in_specs=[
            pl.BlockSpec((NC, BLK, H), lambda i: (0, i, 0)),
            pl.BlockSpec((NC, BLK, CW), lambda i: (0, i, 0)),
            pl.BlockSpec((BLK, H), row),
            pl.BlockSpec((H, H), full), vec,
            pl.BlockSpec((H, H), full), vec, vec,
        ],
        out_specs=pl.BlockSpec((BLK, H), row),
        out_shape=jax.ShapeDtypeStruct((N, H), jnp.float32),
    )(parts, cnt, h, wlt, bl, wrt, g, b)


def kernel(x, edge_index, node_type,
           proc_ln_g, proc_ln_b, proc_w, proc_b,
           file_ln_g, file_ln_b, file_w, file_b,
           sock_ln_g, sock_ln_b, sock_w, sock_b,
           type_emb,
           w_l0, b_l0, w_r0, ln_g0, ln_b0,
           w_l1, b_l1, w_r1, ln_g1, ln_b1):
    f32 = jnp.float32
    nt2 = node_type.reshape(N, 1).astype(jnp.int32)
    emb = jnp.zeros((8, H), f32).at[0:3].set(type_emb)
    r1 = lambda v: v.reshape(1, -1).astype(f32)

    h0 = _proj(x, nt2,
               r1(proc_ln_g), r1(proc_ln_b), proc_w.T, r1(proc_b),
               r1(file_ln_g), r1(file_ln_b), file_w.T, r1(file_b),
               r1(sock_ln_g), r1(sock_ln_b), sock_w.T, r1(sock_b),
               emb)

    src = edge_index[0].astype(jnp.int32)
    dst = edge_index[1].astype(jnp.int32)
    pad = E_PAD - E
    src2 = jnp.concatenate([src, jnp.zeros((pad,), jnp.int32)]
                           ).reshape(TOTCH, CHUNK)
    dst2 = jnp.concatenate([dst, jnp.full((pad,), N, jnp.int32)]
                           ).reshape(TOTCH, CHUNK)
    zrows = jnp.zeros((ROWS_PER_TILE, H), f32)
    ones_arr = jnp.zeros((CHUNK, CW), f32).at[:, 0].set(1.0)

    parts0, cnt0 = _aggregate(h0, src2, dst2, zrows, ones_arr, with_cnt=True)
    h1 = _layer(parts0, cnt0, h0, w_l0.T, r1(b_l0), w_r0.T,
                r1(ln_g0), r1(ln_b0))
    parts1, _ = _aggregate(h1, src2, dst2, zrows, ones_arr, with_cnt=False)
    h2 = _layer(parts1, cnt0, h1, w_l1.T, r1(b_l1), w_r1.T,
                r1(ln_g1), r1(ln_b1))
    return h2


# speedup vs baseline: 1.2544x; 1.0908x over previous
import functools

import jax
import jax.numpy as jnp
from jax import lax
from jax.experimental import pallas as pl
from jax.experimental.pallas import tpu as pltpu
from jax.experimental.pallas import tpu_sc as plsc

N = 10000
D = 128
H = 128
E = 320000

NC = 2
NS = 16
CHUNK = 128
N0_CH = 111
N1_CH = 46
NTOT = E // CHUNK
NP = 10016
ROWS_PER_TILE = NP // NS
CW = 8

BLK = 400
GRID = N // BLK


def _ln(h, g, b):
    m = jnp.mean(h, axis=-1, keepdims=True)
    v = jnp.mean((h - m) * (h - m), axis=-1, keepdims=True)
    return (h - m) / jnp.sqrt(v + 1e-5) * g + b


def _proj_body(x_ref, nt_ref, pg, pbta, pwt, pbi, fg, fb, fwt, fbi,
               sg, sb, swt, sbi, emb_ref, out_ref):
    x = jnp.clip(x_ref[...], -10.0, 10.0)
    p = jnp.dot(_ln(x, pg[...], pbta[...]), pwt[...],
                preferred_element_type=jnp.float32) + pbi[...]
    f = jnp.dot(_ln(x, fg[...], fb[...]), fwt[...],
                preferred_element_type=jnp.float32) + fbi[...]
    s = jnp.dot(_ln(x, sg[...], sb[...]), swt[...],
                preferred_element_type=jnp.float32) + sbi[...]
    nt = nt_ref[...]
    sel = jnp.where(nt == 0, p, jnp.where(nt == 1, f,
                    jnp.where(nt == 2, s, 0.0)))
    te = jnp.where(nt == 0, emb_ref[0:1, :], jnp.where(
        nt == 1, emb_ref[1:2, :], emb_ref[2:3, :]))
    out_ref[...] = sel + te


def _proj(x, nt2, pg, pbta, pwt, pbi, fg, fb, fwt, fbi, sg, sb, swt, sbi, emb):
    row = lambda i: (i, 0)
    full = lambda i: (0, 0)
    vec = pl.BlockSpec((1, H), full)
    return pl.pallas_call(
        _proj_body,
        grid=(GRID,),
        in_specs=[
            pl.BlockSpec((BLK, D), row),
            pl.BlockSpec((BLK, 1), row),
            vec, vec, pl.BlockSpec((D, H), full), vec,
            vec, vec, pl.BlockSpec((D, H), full), vec,
            vec, vec, pl.BlockSpec((D, H), full), vec,
            pl.BlockSpec((8, H), full),
        ],
        out_specs=pl.BlockSpec((BLK, H), row),
        out_shape=jax.ShapeDtypeStruct((N, H), jnp.float32),
    )(x, nt2, pg, pbta, pwt, pbi, fg, fb, fwt, fbi, sg, sb, swt, sbi, emb)


def _agg_body(h_hbm, src_hbm, dst_hbm, zero_hbm, ones_hbm, out_hbm, cnt_hbm,
              srcs, dsts, bufs, ones_v, acc_sh, cnt_sh,
              isems, gsems, ssems, semc, *, with_cnt):
    c = lax.axis_index("c")
    s = lax.axis_index("s")
    nch = jnp.where(c == 0, N0_CH, N1_CH)
    cbase = c * NS * N0_CH + s * nch

    base = s * ROWS_PER_TILE
    pltpu.sync_copy(zero_hbm.at[pl.ds(0, ROWS_PER_TILE)],
                    acc_sh.at[pl.ds(base, ROWS_PER_TILE)])
    if with_cnt:
        pltpu.sync_copy(zero_hbm.at[pl.ds(0, ROWS_PER_TILE), pl.ds(0, CW)],
                        cnt_sh.at[pl.ds(base, ROWS_PER_TILE)])
        pltpu.sync_copy(ones_hbm, ones_v)
    plsc.subcore_barrier()

    def start_idx(i, q):
        pltpu.async_copy(src_hbm.at[cbase + i], srcs[q], isems[q])
        pltpu.async_copy(dst_hbm.at[cbase + i], dsts[q], isems[q])

    def wait_idx(i, q):
        pltpu.make_async_copy(src_hbm.at[cbase + i], srcs[q], isems[q]).wait()
        pltpu.make_async_copy(dst_hbm.at[cbase + i], dsts[q], isems[q]).wait()

    def start_gather(q, b):
        pltpu.async_copy(h_hbm.at[srcs[q]], bufs[b], gsems[b])

    def wait_gather(q, b):
        pltpu.make_async_copy(h_hbm.at[srcs[q]], bufs[b], gsems[b]).wait()

    def start_scat(q, b):
        pltpu.async_copy(bufs[b], acc_sh.at[dsts[q]], ssems[b], add=True)

    def wait_scat(q, b):
        pltpu.make_async_copy(bufs[b], acc_sh.at[dsts[q]], ssems[b]).wait()

    pltpu.sync_copy(src_hbm.at[cbase + 0], srcs[0])
    pltpu.sync_copy(dst_hbm.at[cbase + 0], dsts[0])
    pltpu.sync_copy(src_hbm.at[cbase + 1], srcs[1])
    pltpu.sync_copy(dst_hbm.at[cbase + 1], dsts[1])
    start_idx(2, 2)
    start_idx(3, 3)
    start_gather(0, 0)
    start_gather(1, 1)

    def body(k, _):
        for j in range(4):
            i = 4 * k + j
            q = j
            b = j % 2
            @pl.when(i < nch)
            def _():
                wait_gather(q, b)
                @pl.when(cbase + i < NTOT)
                def _():
                    if with_cnt:
                        pltpu.async_copy(ones_v, cnt_sh.at[dsts[q]], semc,
                                         add=True)
                    pltpu.sync_copy(bufs[b], acc_sh.at[dsts[q]], add=True)
                    if with_cnt:
                        pltpu.make_async_copy(ones_v, cnt_sh.at[dsts[q]],
                                              semc).wait()
                @pl.when((i + 4 < nch) & (cbase + i + 4 < NTOT))
                def _():
                    start_idx(i + 4, q)
                @pl.when(i + 2 < nch)
                def _():
                    q2 = (j + 2) % 4
                    @pl.when(cbase + i + 2 < NTOT)
                    def _():
                        wait_idx(i + 2, q2)
                    start_gather(q2, b)
        return 0
    lax.fori_loop(0, (nch + 3) // 4, body, 0)

    plsc.subcore_barrier()
    pltpu.sync_copy(acc_sh.at[pl.ds(base, ROWS_PER_TILE)],
                    out_hbm.at[c, pl.ds(base, ROWS_PER_TILE)])
    if with_cnt:
        pltpu.sync_copy(cnt_sh.at[pl.ds(base, ROWS_PER_TILE)],
                        cnt_hbm.at[c, pl.ds(base, ROWS_PER_TILE),
                                   pl.ds(0, CW)])


def _aggregate(h, src2, dst2, zrows, ones_arr, with_cnt):
    mesh = plsc.VectorSubcoreMesh(core_axis_name="c", subcore_axis_name="s",
                                  num_cores=NC, num_subcores=NS)
    out_type = (jax.ShapeDtypeStruct((NC, NP, H), jnp.float32),
                jax.ShapeDtypeStruct((NC, NP, H), jnp.float32))
    idx_t = pltpu.VMEM((CHUNK,), jnp.int32)
    buf_t = pltpu.VMEM((CHUNK, H), jnp.float32)
    sem = pltpu.SemaphoreType.DMA
    return pl.kernel(
        functools.partial(_agg_body, with_cnt=with_cnt),
        out_type=out_type,
        mesh=mesh,
        compiler_params=pltpu.CompilerParams(use_tc_tiling_on_sc=False),
        scratch_types=[
            (idx_t,) * 4,
            (idx_t,) * 4,
            (buf_t,) * 2,
            pltpu.VMEM((CHUNK, CW), jnp.float32),
            pltpu.VMEM_SHARED((NP, H), jnp.float32),
            pltpu.VMEM_SHARED((NP, CW), jnp.float32),
            (sem,) * 4,
            (sem,) * 2,
            (sem,) * 2,
            sem,
        ],
    )(h, src2, dst2, zrows, ones_arr)


def _layer_body(parts_ref, cnt_ref, h_ref, wlt, bl, wrt, g, b, out_ref):
    sums = parts_ref[0] + parts_ref[1]
    cnt = cnt_ref[0][:, 0:1] + cnt_ref[1][:, 0:1]
    agg = sums / jnp.maximum(cnt, 1.0)
    h = h_ref[...]
    t = (jnp.dot(agg, wlt[...], preferred_element_type=jnp.float32)
         + jnp.dot(h, wrt[...], preferred_element_type=jnp.float32)
         + bl[...] + h)
    out_ref[...] = jax.nn.relu(_ln(t, g[...], b[...]))


def _layer(parts, cnt, h, wlt, bl, wrt, g, b):
    row = lambda i: (i, 0)
    full = lambda i: (0, 0)
    vec = pl.BlockSpec((1, H), full)
    return pl.pallas_call(
        _layer_body,
        grid=(GRID,),
        in_specs=[
            pl.BlockSpec((NC, BLK, H), lambda i: (0, i, 0)),
            pl.BlockSpec((NC, BLK, H), lambda i: (0, i, 0)),
            pl.BlockSpec((BLK, H), row),
            pl.BlockSpec((H, H), full), vec,
            pl.BlockSpec((H, H), full), vec, vec,
        ],
        out_specs=pl.BlockSpec((BLK, H), row),
        out_shape=jax.ShapeDtypeStruct((N, H), jnp.float32),
    )(parts, cnt, h, wlt, bl, wrt, g, b)


def kernel(x, edge_index, node_type,
           proc_ln_g, proc_ln_b, proc_w, proc_b,
           file_ln_g, file_ln_b, file_w, file_b,
           sock_ln_g, sock_ln_b, sock_w, sock_b,
           type_emb,
           w_l0, b_l0, w_r0, ln_g0, ln_b0,
           w_l1, b_l1, w_r1, ln_g1, ln_b1):
    f32 = jnp.float32
    nt2 = node_type.reshape(N, 1).astype(jnp.int32)
    emb = jnp.zeros((8, H), f32).at[0:3].set(type_emb)
    r1 = lambda v: v.reshape(1, -1).astype(f32)

    h0 = _proj(x, nt2,
               r1(proc_ln_g), r1(proc_ln_b), proc_w.T, r1(proc_b),
               r1(file_ln_g), r1(file_ln_b), file_w.T, r1(file_b),
               r1(sock_ln_g), r1(sock_ln_b), sock_w.T, r1(sock_b),
               emb)

    src2 = edge_index[0].astype(jnp.int32).reshape(NTOT, CHUNK)
    dst2 = edge_index[1].astype(jnp.int32).reshape(NTOT, CHUNK)
    zrows = jnp.zeros((ROWS_PER_TILE, H), f32)
    ones_arr = jnp.zeros((CHUNK, CW), f32).at[:, 0].set(1.0)

    parts0, cnt0 = _aggregate(h0, src2, dst2, zrows, ones_arr, with_cnt=True)
    h1 = _layer(parts0, cnt0, h0, w_l0.T, r1(b_l0), w_r0.T,
                r1(ln_g0), r1(ln_b0))
    parts1, _ = _aggregate(h1, src2, dst2, zrows, ones_arr, with_cnt=False)
    h2 = _layer(parts1, cnt0, h1, w_l1.T, r1(b_l1), w_r1.T,
                r1(ln_g1), r1(ln_b1))
    return h2
